# Initial kernel scaffold; baseline (speedup 1.0000x reference)
import jax
import jax.numpy as jnp
from jax.experimental import pallas as pl


def kernel(x, edge_index, W_gat, att_src, att_dst, bias_gat, w_ih1, w_hh1, b_ih1, b_hh1, w_ih2, w_hh2, b_ih2, b_hh2, lin_w, lin_b):
    raise NotImplementedError("write your pallas kernel here")


# speedup vs baseline: 1.2149x; 1.2149x over previous
import functools
import jax
import jax.numpy as jnp
from jax import lax
from jax.experimental import pallas as pl
from jax.experimental.pallas import tpu as pltpu
from jax.experimental.pallas import tpu_sc as plsc

B = 2
N_NODES = 10000
N = 20000
C = 128
H = 8
E_REAL = 340000
NC, NS, L = 2, 16, 16
NW = NC * NS
SUB = 128
K_SUB = 84
E_PAD = NW * SUB * K_SUB
N_PAD = 20224
NT = N + 8
WT = N_NODES + 8
EB = 512


def _k1_body(x_ref, wg_ref, asrc_ref, adst_ref, s_ref, m_ref, acc_ref):
    i = pl.program_id(0)
    w3 = wg_ref[...].reshape(C, H, C)
    As = jnp.sum(w3 * asrc_ref[...][None], axis=-1)
    Ad = jnp.sum(w3 * adst_ref[...][None], axis=-1)
    AA = jnp.concatenate([As, Ad], axis=1)
    s = jnp.dot(x_ref[...], AA, preferred_element_type=jnp.float32)
    s_ref[...] = s
    ms = jnp.max(s[:, :H])
    md = jnp.max(s[:, H:])

    @pl.when(i == 0)
    def _():
        acc_ref[0] = ms
        acc_ref[1] = md

    @pl.when(i > 0)
    def _():
        acc_ref[0] = jnp.maximum(acc_ref[0], ms)
        acc_ref[1] = jnp.maximum(acc_ref[1], md)

    @pl.when(i == pl.num_programs(0) - 1)
    def _():
        v = jnp.maximum(acc_ref[0] + acc_ref[1], 0.0)
        m_ref[...] = jnp.full((1, 1), v, jnp.float32)


def _k1(x_pad, wg, asrc, adst):
    nblk = N_PAD // 256
    return pl.pallas_call(
        _k1_body,
        grid=(nblk,),
        in_specs=[
            pl.BlockSpec((256, C), lambda i: (i, 0)),
            pl.BlockSpec((C, H * C), lambda i: (0, 0)),
            pl.BlockSpec((H, C), lambda i: (0, 0)),
            pl.BlockSpec((H, C), lambda i: (0, 0)),
        ],
        out_specs=[
            pl.BlockSpec((256, 16), lambda i: (i, 0)),
            pl.BlockSpec((1, 1), lambda i: (0, 0)),
        ],
        out_shape=[
            jax.ShapeDtypeStruct((N_PAD, 16), jnp.float32),
            jax.ShapeDtypeStruct((1, 1), jnp.float32),
        ],
        scratch_shapes=[pltpu.SMEM((2,), jnp.float32)],
    )(x_pad, wg, asrc, adst)


def _k2_body(src_hbm, dst_hbm, tabs_hbm, tabd_hbm, xtab_hbm, w1t_hbm,
             m_hbm,
             ea_hbm, xs_hbm, g_hbm,
             isv, idv, dlv, rsv, rdv, eav, xsv, gv, mv, sem):
    cid = lax.axis_index("c")
    sid = lax.axis_index("s")
    wid = sid * NC + cid

    pltpu.sync_copy(m_hbm, mv)
    mvec = mv[...]

    def step(k, carry):
        base = wid * (SUB * K_SUB) + k * SUB
        pltpu.sync_copy(src_hbm.at[pl.ds(base, SUB)], isv)
        pltpu.sync_copy(dst_hbm.at[pl.ds(base, SUB)], idv)
        pltpu.async_copy(tabs_hbm.at[isv], rsv, sem).wait()
        pltpu.async_copy(tabd_hbm.at[idv], rdv, sem).wait()
        pltpu.async_copy(xtab_hbm.at[isv], xsv, sem).wait()

        def dloc(j, c2):
            d = idv[pl.ds(j * L, L)]
            dlv[pl.ds(j * L, L)] = jnp.where(d >= N_NODES, d - N_NODES, d)
            return c2
        lax.fori_loop(0, SUB // L, dloc, 0)

        pltpu.async_copy(w1t_hbm.at[dlv], gv, sem).wait()

        def edge(e, c2):
            a = rsv[e, 0:L] + rdv[e, 0:L]
            a = jnp.where(a > 0, a, 0.2 * a)
            eav[e, :] = jnp.exp(a - mvec)
            return c2
        lax.fori_loop(0, SUB, edge, 0)

        pltpu.sync_copy(eav, ea_hbm.at[pl.ds(base, SUB)])
        pltpu.sync_copy(xsv, xs_hbm.at[pl.ds(base, SUB)])
        pltpu.sync_copy(gv, g_hbm.at[pl.ds(base, SUB)])
        return carry

    lax.fori_loop(0, K_SUB, step, 0)


def _k2(src, dst, tabs, tabd, xtab, w1t, m_arr):
    mesh = plsc.VectorSubcoreMesh(core_axis_name="c", subcore_axis_name="s")
    fn = pl.kernel(
        _k2_body,
        mesh=mesh,
        out_type=[
            jax.ShapeDtypeStruct((E_PAD, 16), jnp.float32),
            jax.ShapeDtypeStruct((E_PAD, C), jnp.float32),
            jax.ShapeDtypeStruct((E_PAD, C), jnp.float32),
        ],
        scratch_types=[
            pltpu.VMEM((SUB,), jnp.int32),
            pltpu.VMEM((SUB,), jnp.int32),
            pltpu.VMEM((SUB,), jnp.int32),
            pltpu.VMEM((SUB, C), jnp.float32),
            pltpu.VMEM((SUB, C), jnp.float32),
            pltpu.VMEM((SUB, 16), jnp.float32),
            pltpu.VMEM((SUB, C), jnp.float32),
            pltpu.VMEM((SUB, C), jnp.float32),
            pltpu.VMEM((L,), jnp.float32),
            pltpu.SemaphoreType.DMA,
        ],
    )
    return fn(src, dst, tabs, tabd, xtab, w1t, m_arr)


def _k2b_body(ea_ref, dstm_ref, den_ref):
    j = pl.program_id(0)
    i = pl.program_id(1)
    rows = j * 256 + lax.broadcasted_iota(jnp.int32, (256, 1), 0)
    mask = (rows == dstm_ref[...]).astype(jnp.float32)
    contrib = jnp.dot(mask, ea_ref[...], preferred_element_type=jnp.float32)

    @pl.when(i == 0)
    def _():
        den_ref[...] = contrib

    @pl.when(i > 0)
    def _():
        den_ref[...] = den_ref[...] + contrib


def _k2b(ea, dst_mat):
    nj = N_PAD // 256
    ni = E_PAD // EB
    return pl.pallas_call(
        _k2b_body,
        grid=(nj, ni),
        in_specs=[
            pl.BlockSpec((EB, 16), lambda j, i: (i, 0)),
            pl.BlockSpec((1, EB), lambda j, i: (0, i)),
        ],
        out_specs=pl.BlockSpec((256, 16), lambda j, i: (j, 0)),
        out_shape=jax.ShapeDtypeStruct((N_PAD, 16), jnp.float32),
    )(ea, dst_mat)


def _k3_body(dst_hbm, d0_hbm, dn_hbm, idv, r0, dnv, sem):
    cid = lax.axis_index("c")
    sid = lax.axis_index("s")
    wid = sid * NC + cid

    def step(k, carry):
        base = wid * (SUB * K_SUB) + k * SUB
        pltpu.sync_copy(dst_hbm.at[pl.ds(base, SUB)], idv)
        pltpu.async_copy(d0_hbm.at[idv], r0, sem).wait()

        def edge(e, c2):
            dnv[e, :] = r0[e, 0:L]
            return c2
        lax.fori_loop(0, SUB, edge, 0)
        pltpu.sync_copy(dnv, dn_hbm.at[pl.ds(base, SUB)])
        return carry

    lax.fori_loop(0, K_SUB, step, 0)


def _k3(dst, den0):
    mesh = plsc.VectorSubcoreMesh(core_axis_name="c", subcore_axis_name="s")
    fn = pl.kernel(
        _k3_body,
        mesh=mesh,
        out_type=[jax.ShapeDtypeStruct((E_PAD, 16), jnp.float32)],
        scratch_types=[
            pltpu.VMEM((SUB,), jnp.int32),
            pltpu.VMEM((SUB, C), jnp.float32),
            pltpu.VMEM((SUB, 16), jnp.float32),
            pltpu.SemaphoreType.DMA,
        ],
    )
    return fn(dst, den0)[0]


def _k4_body(xs_ref, g_ref, ea_ref, dn_ref, dv_ref, wg_ref, acc_ref):
    i = pl.program_id(0)
    coef = ea_ref[...][:, :H] / (dn_ref[...][:, :H] + 1e-16) * 0.125
    T = jnp.dot(xs_ref[...], wg_ref[...], preferred_element_type=jnp.float32)
    V = coef[:, 0:1] * T[:, 0:C]
    for h in range(1, H):
        V = V + coef[:, h:h + 1] * T[:, h * C:(h + 1) * C]
    m0 = (dv_ref[...] < N_NODES).astype(jnp.float32)
    G = g_ref[...]
    dn = (((0,), (0,)), ((), ()))
    A0 = lax.dot_general(G * m0, V, dn, preferred_element_type=jnp.float32)
    A1 = lax.dot_general(G * (1.0 - m0), V, dn,
                         preferred_element_type=jnp.float32)
    new = jnp.concatenate([A0, A1], axis=0)

    @pl.when(i == 0)
    def _():
        acc_ref[...] = new

    @pl.when(i > 0)
    def _():
        acc_ref[...] = acc_ref[...] + new


def _k4(xs, g, ea, dn, dv, wg):
    nblk = E_PAD // EB
    return pl.pallas_call(
        _k4_body,
        grid=(nblk,),
        in_specs=[
            pl.BlockSpec((EB, C), lambda i: (i, 0)),
            pl.BlockSpec((EB, C), lambda i: (i, 0)),
            pl.BlockSpec((EB, 16), lambda i: (i, 0)),
            pl.BlockSpec((EB, 16), lambda i: (i, 0)),
            pl.BlockSpec((EB, 1), lambda i: (i, 0)),
            pl.BlockSpec((C, H * C), lambda i: (0, 0)),
        ],
        out_specs=pl.BlockSpec((2 * C, C), lambda i: (0, 0)),
        out_shape=jax.ShapeDtypeStruct((2 * C, C), jnp.float32),
    )(xs, g, ea, dn, dv, wg)


def _k5a_body(projT_ref, bg_ref, w1_ref, whh1_ref, wih2_ref, whh2_ref,
              b1_ref, b2_ref, h2_ref):
    rs = jnp.sum(w1_ref[...], axis=1)[None, :]

    def step(t, carry):
        h1, c1, h2, c2 = carry
        pj = projT_ref[t] + bg_ref[t] * rs
        dn1 = (((1,), (1,)), ((), ()))
        g1 = pj + lax.dot_general(h1, whh1_ref[...], dn1,
                                  preferred_element_type=jnp.float32)
        g1 = g1 + b1_ref[...]
        ii, ff, gg, oo = (g1[:, 0:32], g1[:, 32:64], g1[:, 64:96],
                          g1[:, 96:128])
        c1 = jax.nn.sigmoid(ff) * c1 + jax.nn.sigmoid(ii) * jnp.tanh(gg)
        h1 = jax.nn.sigmoid(oo) * jnp.tanh(c1)
        g2 = lax.dot_general(h1, wih2_ref[...], dn1,
                             preferred_element_type=jnp.float32)
        g2 = g2 + lax.dot_general(h2, whh2_ref[...], dn1,
                                  preferred_element_type=jnp.float32)
        g2 = g2 + b2_ref[...]
        i2, f2, gg2, o2 = (g2[:, 0:128], g2[:, 128:256], g2[:, 256:384],
                           g2[:, 384:512])
        c2 = jax.nn.sigmoid(f2) * c2 + jax.nn.sigmoid(i2) * jnp.tanh(gg2)
        h2 = jax.nn.sigmoid(o2) * jnp.tanh(c2)
        return (h1, c1, h2, c2)

    z32 = jnp.zeros((B, 32), jnp.float32)
    z128 = jnp.zeros((B, C), jnp.float32)
    _, _, h2, _ = lax.fori_loop(0, C, step, (z32, z32, z128, z128))
    h2_ref[...] = h2


def _k5a(projT, bg, w1, whh1, wih2, whh2, b1, b2):
    return pl.pallas_call(
        _k5a_body,
        in_specs=[
            pl.BlockSpec((C, B, C), lambda: (0, 0, 0)),
            pl.BlockSpec(memory_space=pltpu.SMEM),
            pl.BlockSpec((C, N_NODES), lambda: (0, 0)),
            pl.BlockSpec((4 * 32, 32), lambda: (0, 0)),
            pl.BlockSpec((4 * C, 32), lambda: (0, 0)),
            pl.BlockSpec((4 * C, C), lambda: (0, 0)),
            pl.BlockSpec((1, 4 * 32), lambda: (0, 0)),
            pl.BlockSpec((1, 4 * C), lambda: (0, 0)),
        ],
        out_specs=pl.BlockSpec((B, C), lambda: (0, 0)),
        out_shape=jax.ShapeDtypeStruct((B, C), jnp.float32),
    )(projT, bg, w1, whh1, wih2, whh2, b1, b2)


def _k5b_body(h_ref, lw_ref, lb_ref, y_ref):
    dn1 = (((1,), (1,)), ((), ()))
    y_ref[...] = lax.dot_general(h_ref[...], lw_ref[...], dn1,
                                 preferred_element_type=jnp.float32) + lb_ref[...]


def _k5b(h2p, lw_pad, lb_pad, mpad):
    nblk = mpad // 1024
    return pl.pallas_call(
        _k5b_body,
        grid=(nblk,),
        in_specs=[
            pl.BlockSpec((8, C), lambda i: (0, 0)),
            pl.BlockSpec((1024, C), lambda i: (i, 0)),
            pl.BlockSpec((1, 1024), lambda i: (0, i)),
        ],
        out_specs=pl.BlockSpec((8, 1024), lambda i: (0, i)),
        out_shape=jax.ShapeDtypeStruct((8, mpad), jnp.float32),
    )(h2p, lw_pad, lb_pad)


@jax.jit
def kernel(x, edge_index, W_gat, att_src, att_dst, bias_gat, w_ih1, w_hh1,
           b_ih1, b_hh1, w_ih2, w_hh2, b_ih2, b_hh2, lin_w, lin_b):
    loop = jnp.arange(N, dtype=edge_index.dtype)
    src = jnp.concatenate([edge_index[0], loop])
    dst = jnp.concatenate([edge_index[1], loop])
    src = jnp.pad(src, (0, E_PAD - E_REAL), constant_values=N)
    dst = jnp.pad(dst, (0, E_PAD - E_REAL), constant_values=N)

    x_pad = jnp.pad(x, ((0, N_PAD - N), (0, 0)))
    s_pad, m = _k1(x_pad, W_gat, att_src, att_dst)
    tabs = jnp.pad(s_pad[:N, :H], ((0, NT - N), (0, C - H)))
    tabd = jnp.pad(s_pad[:N, H:], ((0, NT - N), (0, C - H)))
    m_arr = jnp.broadcast_to(m[0, 0][None], (L,))
    x_tab = jnp.pad(x, ((0, NT - N), (0, 0)))
    w1t = jnp.pad(jnp.transpose(w_ih1), ((0, WT - N_NODES), (0, 0)))

    ea, xs, g = _k2(src, dst, tabs, tabd, x_tab, w1t, m_arr)
    denom = _k2b(ea, dst.reshape(1, E_PAD))
    den_pad = jnp.pad(denom[:NT], ((0, 0), (0, C - 16)))
    dn = _k3(dst, den_pad)

    acc = _k4(xs, g, ea, dn, dst.reshape(E_PAD, 1), W_gat)
    projT = jnp.transpose(acc.reshape(B, C, C), (2, 0, 1))

    b1 = (b_ih1 + b_hh1).reshape(1, 4 * 32)
    b2 = (b_ih2 + b_hh2).reshape(1, 4 * C)
    h2 = _k5a(projT, bias_gat, w_ih1, w_hh1, w_ih2, w_hh2, b1, b2)

    mpad = 90112
    h2p = jnp.pad(h2, ((0, 8 - B), (0, 0)))
    lw_pad = jnp.pad(lin_w, ((0, mpad - N_NODES * 9), (0, 0)))
    lb_pad = jnp.pad(lin_b, (0, mpad - N_NODES * 9)).reshape(1, mpad)
    y = _k5b(h2p, lw_pad, lb_pad, mpad)
    return y[:B, :N_NODES * 9].reshape(N, 9)
